# Initial kernel scaffold; baseline (speedup 1.0000x reference)
import jax
import jax.numpy as jnp
from jax.experimental import pallas as pl


def kernel(layer_idx, hidden, W):
    raise NotImplementedError("write your pallas kernel here")


# speedup vs baseline: 1.1251x; 1.1251x over previous
import functools

import jax
import jax.numpy as jnp
from jax.experimental import pallas as pl
from jax.experimental.pallas import tpu as pltpu

N_TOK = 32768
D_MODEL = 4096
N_EXP = 64
K_TOP = 8
BLK = 512


def _router_body(h_ref, w_ref, idx_ref, wgt_ref, logit_ref):
    h = h_ref[...]
    w = w_ref[...]
    logits = jax.lax.dot_general(
        h, w, (((1,), (1,)), ((), ())),
        preferred_element_type=jnp.float32)
    logit_ref[...] = logits

    m = jnp.max(logits, axis=-1, keepdims=True)
    ex = jnp.exp(logits - m)
    probs = ex / jnp.sum(ex, axis=-1, keepdims=True)

    cur = probs
    lane = jax.lax.broadcasted_iota(jnp.int32, cur.shape, 1)
    idx_cols = []
    wgt_cols = []
    for _ in range(K_TOP):
        mx = jnp.max(cur, axis=-1, keepdims=True)
        amax = jnp.min(jnp.where(cur == mx, lane, N_EXP),
                       axis=-1, keepdims=True)
        idx_cols.append(amax)
        wgt_cols.append(mx)
        cur = jnp.where(lane == amax, -1.0, cur)
    idx_ref[...] = jnp.concatenate(idx_cols, axis=1)
    wgt_ref[...] = jnp.concatenate(wgt_cols, axis=1)


def kernel(layer_idx, hidden, W):
    n_tok = hidden.shape[0]
    grid = (n_tok // BLK,)
    out_shapes = (
        jax.ShapeDtypeStruct((n_tok, K_TOP), jnp.int32),
        jax.ShapeDtypeStruct((n_tok, K_TOP), jnp.float32),
        jax.ShapeDtypeStruct((n_tok, N_EXP), jnp.float32),
    )
    idx, wgt, logits = pl.pallas_call(
        _router_body,
        grid=grid,
        in_specs=[
            pl.BlockSpec((BLK, D_MODEL), lambda i: (i, 0)),
            pl.BlockSpec((N_EXP, D_MODEL), lambda i: (0, 0)),
        ],
        out_specs=(
            pl.BlockSpec((BLK, K_TOP), lambda i: (i, 0)),
            pl.BlockSpec((BLK, K_TOP), lambda i: (i, 0)),
            pl.BlockSpec((BLK, N_EXP), lambda i: (i, 0)),
        ),
        out_shape=out_shapes,
    )(hidden, W)
    return (idx, wgt, logits)


# speedup vs baseline: 1.3669x; 1.2149x over previous
import functools

import jax
import jax.numpy as jnp
from jax.experimental import pallas as pl
from jax.experimental.pallas import tpu as pltpu

N_TOK = 32768
D_MODEL = 4096
N_EXP = 64
K_TOP = 8
BLK = 512


def _router_body(h_ref, w_ref, idx_ref, wgt_ref, logit_ref):
    h = h_ref[...]
    w = w_ref[...]
    lt = jax.lax.dot_general(
        w, h, (((1,), (1,)), ((), ())),
        preferred_element_type=jnp.float32)
    logit_ref[...] = lt.T

    m = jnp.max(lt, axis=0, keepdims=True)
    ex = jnp.exp(lt - m)
    probs = ex / jnp.sum(ex, axis=0, keepdims=True)

    cur = probs
    e_iota = jax.lax.broadcasted_iota(jnp.int32, cur.shape, 0)
    idx_rows = []
    wgt_rows = []
    for _ in range(K_TOP):
        mx = jnp.max(cur, axis=0, keepdims=True)
        amax = jnp.min(jnp.where(cur == mx, e_iota, N_EXP),
                       axis=0, keepdims=True)
        idx_rows.append(amax)
        wgt_rows.append(mx)
        cur = jnp.where(e_iota == amax, -1.0, cur)
    idx_ref[...] = jnp.concatenate(idx_rows, axis=0).T
    wgt_ref[...] = jnp.concatenate(wgt_rows, axis=0).T


def kernel(layer_idx, hidden, W):
    n_tok = hidden.shape[0]
    grid = (n_tok // BLK,)
    out_shapes = (
        jax.ShapeDtypeStruct((n_tok, K_TOP), jnp.int32),
        jax.ShapeDtypeStruct((n_tok, K_TOP), jnp.float32),
        jax.ShapeDtypeStruct((n_tok, N_EXP), jnp.float32),
    )
    idx, wgt, logits = pl.pallas_call(
        _router_body,
        grid=grid,
        in_specs=[
            pl.BlockSpec((BLK, D_MODEL), lambda i: (i, 0)),
            pl.BlockSpec((N_EXP, D_MODEL), lambda i: (0, 0)),
        ],
        out_specs=(
            pl.BlockSpec((BLK, K_TOP), lambda i: (i, 0)),
            pl.BlockSpec((BLK, K_TOP), lambda i: (i, 0)),
            pl.BlockSpec((BLK, N_EXP), lambda i: (i, 0)),
        ),
        out_shape=out_shapes,
    )(hidden, W)
    return (idx, wgt, logits)


# speedup vs baseline: 1.4689x; 1.0746x over previous
import functools

import jax
import jax.numpy as jnp
from jax.experimental import pallas as pl
from jax.experimental.pallas import tpu as pltpu

N_TOK = 32768
D_MODEL = 4096
N_EXP = 64
K_TOP = 8
BLK = 1024


def _router_body(h_ref, w_ref, idx_ref, wgt_ref, logit_ref):
    h = h_ref[...]
    w = w_ref[...]
    lt = jax.lax.dot_general(
        w, h, (((1,), (1,)), ((), ())),
        preferred_element_type=jnp.float32)
    logit_ref[...] = lt.T

    m = jnp.max(lt, axis=0, keepdims=True)
    ex = jnp.exp(lt - m)
    probs = ex / jnp.sum(ex, axis=0, keepdims=True)

    cur = probs
    e_iota = jax.lax.broadcasted_iota(jnp.int32, cur.shape, 0)
    idx_rows = []
    wgt_rows = []
    for _ in range(K_TOP):
        mx = jnp.max(cur, axis=0, keepdims=True)
        amax = jnp.min(jnp.where(cur == mx, e_iota, N_EXP),
                       axis=0, keepdims=True)
        idx_rows.append(amax)
        wgt_rows.append(mx)
        cur = jnp.where(e_iota == amax, -1.0, cur)
    idx_ref[...] = jnp.concatenate(idx_rows, axis=0).T
    wgt_ref[...] = jnp.concatenate(wgt_rows, axis=0).T


def kernel(layer_idx, hidden, W):
    n_tok = hidden.shape[0]
    grid = (n_tok // BLK,)
    out_shapes = (
        jax.ShapeDtypeStruct((n_tok, K_TOP), jnp.int32),
        jax.ShapeDtypeStruct((n_tok, K_TOP), jnp.float32),
        jax.ShapeDtypeStruct((n_tok, N_EXP), jnp.float32),
    )
    idx, wgt, logits = pl.pallas_call(
        _router_body,
        grid=grid,
        in_specs=[
            pl.BlockSpec((BLK, D_MODEL), lambda i: (i, 0)),
            pl.BlockSpec((N_EXP, D_MODEL), lambda i: (0, 0)),
        ],
        out_specs=(
            pl.BlockSpec((BLK, K_TOP), lambda i: (i, 0)),
            pl.BlockSpec((BLK, K_TOP), lambda i: (i, 0)),
            pl.BlockSpec((BLK, N_EXP), lambda i: (i, 0)),
        ),
        out_shape=out_shapes,
    )(hidden, W)
    return (idx, wgt, logits)


# speedup vs baseline: 1.4712x; 1.0016x over previous
import functools

import jax
import jax.numpy as jnp
from jax.experimental import pallas as pl
from jax.experimental.pallas import tpu as pltpu

N_TOK = 32768
D_MODEL = 4096
N_EXP = 64
K_TOP = 8
BLK = 1024


def _router_body(h0_ref, h1_ref, w_ref, idx_ref, wgt_ref, logit_ref):
    half = D_MODEL // 2
    lt = jax.lax.dot_general(
        w_ref[:, :half], h0_ref[...], (((1,), (1,)), ((), ())),
        preferred_element_type=jnp.float32)
    lt = lt + jax.lax.dot_general(
        w_ref[:, half:], h1_ref[...], (((1,), (1,)), ((), ())),
        preferred_element_type=jnp.float32)
    logit_ref[...] = lt.T

    m = jnp.max(lt, axis=0, keepdims=True)
    ex = jnp.exp(lt - m)
    probs = ex / jnp.sum(ex, axis=0, keepdims=True)

    cur = probs
    e_iota = jax.lax.broadcasted_iota(jnp.int32, cur.shape, 0)
    idx_rows = []
    wgt_rows = []
    for _ in range(K_TOP):
        mx = jnp.max(cur, axis=0, keepdims=True)
        amax = jnp.min(jnp.where(cur == mx, e_iota, N_EXP),
                       axis=0, keepdims=True)
        idx_rows.append(amax)
        wgt_rows.append(mx)
        cur = jnp.where(e_iota == amax, -1.0, cur)
    idx_ref[...] = jnp.concatenate(idx_rows, axis=0).T
    wgt_ref[...] = jnp.concatenate(wgt_rows, axis=0).T


def kernel(layer_idx, hidden, W):
    n_tok = hidden.shape[0]
    grid = (n_tok // BLK,)
    out_shapes = (
        jax.ShapeDtypeStruct((n_tok, K_TOP), jnp.int32),
        jax.ShapeDtypeStruct((n_tok, K_TOP), jnp.float32),
        jax.ShapeDtypeStruct((n_tok, N_EXP), jnp.float32),
    )
    idx, wgt, logits = pl.pallas_call(
        _router_body,
        grid=grid,
        in_specs=[
            pl.BlockSpec((BLK, D_MODEL // 2), lambda i: (i, 0)),
            pl.BlockSpec((BLK, D_MODEL // 2), lambda i: (i, 1)),
            pl.BlockSpec((N_EXP, D_MODEL), lambda i: (0, 0)),
        ],
        out_specs=(
            pl.BlockSpec((BLK, K_TOP), lambda i: (i, 0)),
            pl.BlockSpec((BLK, K_TOP), lambda i: (i, 0)),
            pl.BlockSpec((BLK, N_EXP), lambda i: (i, 0)),
        ),
        out_shape=out_shapes,
    )(hidden, hidden, W)
    return (idx, wgt, logits)
